# Initial kernel scaffold; baseline (speedup 1.0000x reference)
import jax
import jax.numpy as jnp
from jax.experimental import pallas as pl


def kernel(states, phases, emb, ln_g, ln_b, W1, b1, Wmu, bmu, Wk, bk):
    raise NotImplementedError("write your pallas kernel here")


# speedup vs baseline: 2.6655x; 2.6655x over previous
import functools

import jax
import jax.numpy as jnp
from jax import lax
from jax.experimental import pallas as pl
from jax.experimental.pallas import tpu as pltpu
from jax.experimental.pallas import tpu_sc as plsc

NSTATES = 32
HID = 64
NTOT = 1048576
NC = 2
NS = 16
L = 16
NW = NC * NS
PER_W = NTOT // NW
NVEC = PER_W // L
ROWS = NSTATES + 2
UNROLL = 8

@functools.cache
def _build_sc_partials():
    mesh = plsc.VectorSubcoreMesh(
        core_axis_name="c", subcore_axis_name="s",
        num_cores=NC, num_subcores=NS)

    @functools.partial(
        pl.kernel,
        out_type=jax.ShapeDtypeStruct((NW, ROWS, L), jnp.float32),
        mesh=mesh,
        scratch_types=[
            pltpu.VMEM((PER_W,), jnp.int32),
            pltpu.VMEM((PER_W,), jnp.float32),
            pltpu.VMEM((ROWS, L), jnp.float32),
        ],
        compiler_params=pltpu.CompilerParams(needs_layout_passes=False),
    )
    def _sc_partials(states_hbm, phases_hbm, out_hbm, st_v, ph_v, blk_v):
        w = lax.axis_index("s") * NC + lax.axis_index("c")
        base = w * PER_W
        pltpu.sync_copy(states_hbm.at[pl.ds(base, PER_W)], st_v)
        pltpu.sync_copy(phases_hbm.at[pl.ds(base, PER_W)], ph_v)

        zero = jnp.zeros((L,), jnp.float32)
        for r in range(ROWS):
            blk_v[r] = zero
        lanes = lax.broadcasted_iota(jnp.int32, (L,), 0)
        ones = jnp.ones((L,), jnp.float32)

        def body(i, carry):
            a1, a2 = carry
            for u in range(UNROLL):
                off = (i * UNROLL + u) * L
                sv = st_v[pl.ds(off, L)]
                pv = ph_v[pl.ds(off, L)]
                plsc.addupdate_scatter(blk_v, [sv, lanes], ones)
                a1 = a1 + pv
                a2 = a2 + pv * pv
            return (a1, a2)

        a1, a2 = lax.fori_loop(0, NVEC // UNROLL, body, (zero, zero))
        blk_v[NSTATES] = a1
        blk_v[NSTATES + 1] = a2
        pltpu.sync_copy(blk_v, out_hbm.at[w])

    return _sc_partials


def _poly(t, coeffs):
    acc = jnp.float32(coeffs[-1])
    for c in coeffs[-2::-1]:
        acc = acc * t + jnp.float32(c)
    return acc


_I0_SMALL = (1.0, 3.5156229, 3.0899424, 1.2067492, 0.2659732,
             0.0360768, 0.0045813)
_I0_LARGE = (0.39894228, 0.01328592, 0.00225319, -0.00157565, 0.00916281,
             -0.02057706, 0.02635537, -0.01647633, 0.00392377)
_I1_SMALL = (0.5, 0.87890594, 0.51498869, 0.15084934, 0.02658733,
             0.00301532, 0.00032411)
_I1_LARGE = (0.39894228, -0.03988024, -0.00362018, 0.00163801, -0.01031555,
             0.02282967, -0.02895312, 0.01787654, -0.00420059)


def _i0e(x):
    xs = jnp.minimum(x, 3.75)
    xl = jnp.maximum(x, 3.75)
    small = _poly((xs / 3.75) ** 2, _I0_SMALL) * jnp.exp(-xs)
    large = _poly(3.75 / xl, _I0_LARGE) / jnp.sqrt(xl)
    return jnp.where(x < 3.75, small, large)


def _i1e(x):
    xs = jnp.minimum(x, 3.75)
    xl = jnp.maximum(x, 3.75)
    small = xs * _poly((xs / 3.75) ** 2, _I1_SMALL) * jnp.exp(-xs)
    large = _poly(3.75 / xl, _I1_LARGE) / jnp.sqrt(xl)
    return jnp.where(x < 3.75, small, large)


def _log_i0e(x):
    return jnp.log(_i0e(x))


LOG_2PI = 1.8378770664093453
N_POW = float(NTOT) ** (-0.2)


def _tc_tail(red_ref, emb_ref, lng_ref, lnb_ref, w1_ref, b1_ref,
             wk_ref, bk_ref, out_ref):
    red = red_ref[...]
    sums = jnp.sum(red, axis=1)
    counts = sums[:NSTATES]
    s1 = sums[NSTATES]
    s2 = sums[NSTATES + 1]
    n = jnp.float32(NTOT)

    probs = counts / n + 1e-10
    h_z = -jnp.sum(probs * jnp.log(probs))

    var_p = (s2 - s1 * s1 / n) / (n - 1.0)
    std_p = jnp.sqrt(jnp.maximum(var_p, 0.0))
    bw = 1.06 * std_p * N_POW
    kap_kde = jnp.minimum(1.0 / (bw * bw + 1e-6), 100.0)
    h_phi = LOG_2PI + _log_i0e(kap_kde) + kap_kde

    h = emb_ref[...]
    mean = jnp.mean(h, axis=1, keepdims=True)
    var = jnp.mean((h - mean) ** 2, axis=1, keepdims=True)
    h = (h - mean) / jnp.sqrt(var + 1e-5) * lng_ref[...] + lnb_ref[...]
    h = jnp.maximum(h, 0.0)
    h = lax.dot_general(h, w1_ref[...], (((1,), (1,)), ((), ())),
                        preferred_element_type=jnp.float32) + b1_ref[...]
    h = jnp.maximum(h, 0.0)
    kp = lax.dot_general(h, wk_ref[...], (((1,), (1,)), ((), ())),
                         preferred_element_type=jnp.float32)[:, 0] + bk_ref[...]
    kappa = jnp.maximum(kp, 0.0) + jnp.log1p(jnp.exp(-jnp.abs(kp))) + 0.1

    i0e_k = _i0e(kappa)
    ratio = _i1e(kappa) / i0e_k
    h_vm = LOG_2PI + jnp.log(i0e_k) + kappa - kappa * ratio
    h_cond = jnp.sum((counts / n) * h_vm)

    mi = h_phi - h_cond
    bdc = jnp.clip(2.0 * mi / (h_z + h_phi + 1e-12), 0.0, 1.0)

    out_ref[0] = mi
    out_ref[1] = h_z
    out_ref[2] = h_phi
    out_ref[3] = h_cond
    out_ref[4] = bdc


def kernel(states, phases, emb, ln_g, ln_b, W1, b1, Wmu, bmu, Wk, bk):
    partials = _build_sc_partials()(states, phases)
    red = jnp.transpose(partials, (1, 0, 2)).reshape(ROWS, NW * L)
    out = pl.pallas_call(
        _tc_tail,
        out_shape=jax.ShapeDtypeStruct((8,), jnp.float32),
        out_specs=pl.BlockSpec(memory_space=pltpu.SMEM),
    )(red, emb, ln_g, ln_b, W1, b1, Wk, bk)
    return (out[0], out[1], out[2], out[3], out[4])


# speedup vs baseline: 3.4948x; 1.3111x over previous
import functools

import jax
import jax.numpy as jnp
from jax import lax
from jax.experimental import pallas as pl
from jax.experimental.pallas import tpu as pltpu
from jax.experimental.pallas import tpu_sc as plsc

NSTATES = 32
HID = 64
NTOT = 1048576
NC = 2
NS = 16
L = 16
NW = NC * NS
PER_W = NTOT // NW
NVEC = PER_W // L
ROWS = NSTATES + 2
UNROLL = 8

@functools.cache
def _build_sc_partials():
    mesh = plsc.VectorSubcoreMesh(
        core_axis_name="c", subcore_axis_name="s",
        num_cores=NC, num_subcores=NS)

    @functools.partial(
        pl.kernel,
        out_type=jax.ShapeDtypeStruct((ROWS, NW, L), jnp.float32),
        mesh=mesh,
        scratch_types=[
            pltpu.VMEM((PER_W,), jnp.int32),
            pltpu.VMEM((PER_W,), jnp.float32),
            pltpu.VMEM((ROWS, L), jnp.float32),
        ],
        compiler_params=pltpu.CompilerParams(needs_layout_passes=False),
    )
    def _sc_partials(states_hbm, phases_hbm, out_hbm, st_v, ph_v, blk_v):
        w = lax.axis_index("s") * NC + lax.axis_index("c")
        base = w * PER_W
        pltpu.sync_copy(states_hbm.at[pl.ds(base, PER_W)], st_v)
        pltpu.sync_copy(phases_hbm.at[pl.ds(base, PER_W)], ph_v)

        zero = jnp.zeros((L,), jnp.float32)
        for r in range(ROWS):
            blk_v[r] = zero
        lanes = lax.broadcasted_iota(jnp.int32, (L,), 0)
        ones = jnp.ones((L,), jnp.float32)

        def body(i, carry):
            (a1e, a2e), (a1o, a2o) = carry
            off = i * L
            sv0 = st_v[pl.ds(off, L)]
            pv0 = ph_v[pl.ds(off, L)]
            plsc.addupdate_scatter(blk_v, [sv0, lanes], ones)
            sv1 = st_v[pl.ds(off + L, L)]
            pv1 = ph_v[pl.ds(off + L, L)]
            plsc.addupdate_scatter(blk_v, [sv1, lanes], ones)
            return ((a1e + pv0, a2e + pv0 * pv0),
                    (a1o + pv1, a2o + pv1 * pv1))

        carry0 = ((zero, zero), (zero, zero))
        (a1e, a2e), (a1o, a2o) = plsc.parallel_loop(
            0, NVEC, 2, unroll=UNROLL, carry=carry0)(body)
        blk_v[NSTATES] = a1e + a1o
        blk_v[NSTATES + 1] = a2e + a2o
        pltpu.sync_copy(blk_v, out_hbm.at[:, w])

    return _sc_partials


def _poly(t, coeffs):
    acc = jnp.float32(coeffs[-1])
    for c in coeffs[-2::-1]:
        acc = acc * t + jnp.float32(c)
    return acc


_I0_SMALL = (1.0, 3.5156229, 3.0899424, 1.2067492, 0.2659732,
             0.0360768, 0.0045813)
_I0_LARGE = (0.39894228, 0.01328592, 0.00225319, -0.00157565, 0.00916281,
             -0.02057706, 0.02635537, -0.01647633, 0.00392377)
_I1_SMALL = (0.5, 0.87890594, 0.51498869, 0.15084934, 0.02658733,
             0.00301532, 0.00032411)
_I1_LARGE = (0.39894228, -0.03988024, -0.00362018, 0.00163801, -0.01031555,
             0.02282967, -0.02895312, 0.01787654, -0.00420059)


def _i0e(x):
    xs = jnp.minimum(x, 3.75)
    xl = jnp.maximum(x, 3.75)
    small = _poly((xs / 3.75) ** 2, _I0_SMALL) * jnp.exp(-xs)
    large = _poly(3.75 / xl, _I0_LARGE) / jnp.sqrt(xl)
    return jnp.where(x < 3.75, small, large)


def _i1e(x):
    xs = jnp.minimum(x, 3.75)
    xl = jnp.maximum(x, 3.75)
    small = xs * _poly((xs / 3.75) ** 2, _I1_SMALL) * jnp.exp(-xs)
    large = _poly(3.75 / xl, _I1_LARGE) / jnp.sqrt(xl)
    return jnp.where(x < 3.75, small, large)


def _log_i0e(x):
    return jnp.log(_i0e(x))


LOG_2PI = 1.8378770664093453
N_POW = float(NTOT) ** (-0.2)


def _tc_tail(red_ref, emb_ref, lng_ref, lnb_ref, w1_ref, b1_ref,
             wk_ref, bk_ref, out_ref):
    red = red_ref[...]
    sums = jnp.sum(red, axis=1)
    counts = sums[:NSTATES]
    s1 = sums[NSTATES]
    s2 = sums[NSTATES + 1]
    n = jnp.float32(NTOT)

    probs = counts / n + 1e-10
    h_z = -jnp.sum(probs * jnp.log(probs))

    var_p = (s2 - s1 * s1 / n) / (n - 1.0)
    std_p = jnp.sqrt(jnp.maximum(var_p, 0.0))
    bw = 1.06 * std_p * N_POW
    kap_kde = jnp.minimum(1.0 / (bw * bw + 1e-6), 100.0)
    h_phi = LOG_2PI + _log_i0e(kap_kde) + kap_kde

    h = emb_ref[...]
    mean = jnp.mean(h, axis=1, keepdims=True)
    var = jnp.mean((h - mean) ** 2, axis=1, keepdims=True)
    h = (h - mean) / jnp.sqrt(var + 1e-5) * lng_ref[...] + lnb_ref[...]
    h = jnp.maximum(h, 0.0)
    h = lax.dot_general(h, w1_ref[...], (((1,), (1,)), ((), ())),
                        preferred_element_type=jnp.float32) + b1_ref[...]
    h = jnp.maximum(h, 0.0)
    kp = lax.dot_general(h, wk_ref[...], (((1,), (1,)), ((), ())),
                         preferred_element_type=jnp.float32)[:, 0] + bk_ref[...]
    kappa = jnp.maximum(kp, 0.0) + jnp.log1p(jnp.exp(-jnp.abs(kp))) + 0.1

    i0e_k = _i0e(kappa)
    ratio = _i1e(kappa) / i0e_k
    h_vm = LOG_2PI + jnp.log(i0e_k) + kappa - kappa * ratio
    h_cond = jnp.sum((counts / n) * h_vm)

    mi = h_phi - h_cond
    bdc = jnp.clip(2.0 * mi / (h_z + h_phi + 1e-12), 0.0, 1.0)

    out_ref[0] = mi
    out_ref[1] = h_z
    out_ref[2] = h_phi
    out_ref[3] = h_cond
    out_ref[4] = bdc


def kernel(states, phases, emb, ln_g, ln_b, W1, b1, Wmu, bmu, Wk, bk):
    partials = _build_sc_partials()(states, phases)
    red = partials.reshape(ROWS, NW * L)
    out = pl.pallas_call(
        _tc_tail,
        out_shape=jax.ShapeDtypeStruct((8,), jnp.float32),
        out_specs=pl.BlockSpec(memory_space=pltpu.SMEM),
    )(red, emb, ln_g, ln_b, W1, b1, Wk, bk)
    return (out[0], out[1], out[2], out[3], out[4])
